# Initial kernel scaffold; baseline (speedup 1.0000x reference)
import jax
import jax.numpy as jnp
from jax.experimental import pallas as pl


def kernel(species_ids, divergence_times, species_table, phylo_table, kingdom_table, phylum_table, class_table, order_table, W, b):
    raise NotImplementedError("write your pallas kernel here")


# speedup vs baseline: 9.4877x; 9.4877x over previous
import functools

import jax
import jax.numpy as jnp
from jax import lax
from jax.experimental import pallas as pl
from jax.experimental.pallas import tpu as pltpu
from jax.experimental.pallas import tpu_sc as plsc

B = 16384
EMB_DIM = 128
PHYLO_DIM = 64
FUSED_IN = 320

_NC = 2
_NS = 16
_NW = _NC * _NS
_BPW = B // _NW
_CH = 128
_NCHUNK = _BPW // _CH


def _sc_gather_body(species_hbm, sid_hbm, s_out, sidx_v, srows_v, sem):
    wid = lax.axis_index("s") * _NC + lax.axis_index("c")
    base = wid * _BPW
    pltpu.sync_copy(sid_hbm.at[pl.ds(base, _BPW)], sidx_v)
    copies = []
    for j in range(_NCHUNK):
        copies.append(pltpu.async_copy(
            species_hbm.at[sidx_v.at[pl.ds(j * _CH, _CH)]],
            srows_v.at[pl.ds(j * _CH, _CH)], sem))
    for c in copies:
        c.wait()
    pltpu.sync_copy(srows_v, s_out.at[pl.ds(base, _BPW)])


@functools.lru_cache(maxsize=None)
def _get_sc_gather():
    return pl.kernel(
        _sc_gather_body,
        out_type=jax.ShapeDtypeStruct((B, EMB_DIM), jnp.float32),
        mesh=plsc.VectorSubcoreMesh(core_axis_name="c", subcore_axis_name="s"),
        scratch_types=[
            pltpu.VMEM((_BPW,), jnp.int32),
            pltpu.VMEM((_BPW, EMB_DIM), jnp.float32),
            pltpu.SemaphoreType.DMA,
        ],
    )


_BLK = 2048


def _tc_fuse_body(s_ref, t_ref, phylo_ref, w_ref, tax_ref, b_ref, out_ref):
    w = w_ref[...]
    ws = w[:, 0:EMB_DIM]
    wp = w[:, EMB_DIM:EMB_DIM + PHYLO_DIM]
    wt = w[:, EMB_DIM + PHYLO_DIM:FUSED_IN]
    dn = (((1,), (1,)), ((), ()))
    c = lax.dot_general(tax_ref[...], wt, dn,
                        preferred_element_type=jnp.float32) + b_ref[...]
    p = lax.dot_general(phylo_ref[...], wp, dn,
                        preferred_element_type=jnp.float32)
    oh = (t_ref[...] == lax.broadcasted_iota(jnp.int32, (_BLK, EMB_DIM), 1)
          ).astype(jnp.float32)
    acc = lax.dot_general(s_ref[...], ws, dn,
                          preferred_element_type=jnp.float32)
    acc += lax.dot_general(oh, p, (((1,), (0,)), ((), ())),
                           preferred_element_type=jnp.float32)
    out_ref[...] = acc + c


_tc_fuse = pl.pallas_call(
    _tc_fuse_body,
    grid=(B // _BLK,),
    in_specs=[
        pl.BlockSpec((_BLK, EMB_DIM), lambda i: (i, 0)),
        pl.BlockSpec((_BLK, 1), lambda i: (i, 0)),
        pl.BlockSpec((EMB_DIM, PHYLO_DIM), lambda i: (0, 0)),
        pl.BlockSpec((EMB_DIM, FUSED_IN), lambda i: (0, 0)),
        pl.BlockSpec((1, EMB_DIM), lambda i: (0, 0)),
        pl.BlockSpec((1, EMB_DIM), lambda i: (0, 0)),
    ],
    out_specs=pl.BlockSpec((_BLK, EMB_DIM), lambda i: (i, 0)),
    out_shape=jax.ShapeDtypeStruct((B, EMB_DIM), jnp.float32),
)


def kernel(species_ids, divergence_times, species_table, phylo_table,
           kingdom_table, phylum_table, class_table, order_table, W, b):
    s_rows = _get_sc_gather()(species_table, species_ids.astype(jnp.int32))
    phylo_pad = jnp.pad(phylo_table, ((0, EMB_DIM - phylo_table.shape[0]),
                                      (0, 0)))
    tax = jnp.concatenate([kingdom_table[0], phylum_table[0],
                           class_table[0], order_table[0]])[None, :]
    times = divergence_times.astype(jnp.int32)[:, None]
    return _tc_fuse(s_rows, times, phylo_pad, W, tax, b[None, :])


# speedup vs baseline: 10.4798x; 1.1046x over previous
import functools

import jax
import jax.numpy as jnp
from jax import lax
from jax.experimental import pallas as pl
from jax.experimental.pallas import tpu as pltpu
from jax.experimental.pallas import tpu_sc as plsc

B = 16384
EMB_DIM = 128
PHYLO_DIM = 64
FUSED_IN = 320

_NC = 2
_NS = 16
_NW = _NC * _NS
_BPW = B // _NW
_CH = 128
_NCHUNK = _BPW // _CH


def _sc_gather_body(species_hbm, sid_hbm, s_out, sidx_v, srows_v, sem):
    wid = lax.axis_index("s") * _NC + lax.axis_index("c")
    base = wid * _BPW
    pltpu.sync_copy(sid_hbm.at[pl.ds(base, _BPW)], sidx_v)
    copies = []
    for j in range(_NCHUNK):
        copies.append(pltpu.async_copy(
            species_hbm.at[sidx_v.at[pl.ds(j * _CH, _CH)]],
            srows_v.at[pl.ds(j * _CH, _CH)], sem))
    for c in copies:
        c.wait()
    pltpu.sync_copy(srows_v, s_out.at[pl.ds(base, _BPW)])


@functools.lru_cache(maxsize=None)
def _get_sc_gather():
    return pl.kernel(
        _sc_gather_body,
        out_type=jax.ShapeDtypeStruct((B, EMB_DIM), jnp.float32),
        mesh=plsc.VectorSubcoreMesh(core_axis_name="c", subcore_axis_name="s"),
        scratch_types=[
            pltpu.VMEM((_BPW,), jnp.int32),
            pltpu.VMEM((_BPW, EMB_DIM), jnp.float32),
            pltpu.SemaphoreType.DMA,
        ],
    )


_BLK = 2048


def _tc_fuse_body(s_ref, t_ref, phylo_ref, k_ref, p_ref, c_ref, o_ref,
                  w_ref, b_ref, out_ref):
    w = w_ref[...]
    ws = w[:, 0:EMB_DIM]
    wp = w[:, EMB_DIM:EMB_DIM + PHYLO_DIM]
    wt = w[:, EMB_DIM + PHYLO_DIM:FUSED_IN]
    dn = (((1,), (1,)), ((), ()))
    tax = jnp.concatenate([k_ref[0:1, :], p_ref[0:1, :],
                           c_ref[0:1, :], o_ref[0:1, :]], axis=1)
    c = lax.dot_general(tax, wt, dn,
                        preferred_element_type=jnp.float32) + b_ref[...]
    p = lax.dot_general(phylo_ref[...], wp, dn,
                        preferred_element_type=jnp.float32)
    oht = (t_ref[0] == lax.broadcasted_iota(jnp.int32, (EMB_DIM, _BLK), 0)
           ).astype(jnp.float32)
    acc = lax.dot_general(s_ref[...], ws, dn,
                          preferred_element_type=jnp.float32)
    acc += lax.dot_general(oht, p, (((0,), (0,)), ((), ())),
                           preferred_element_type=jnp.float32)
    out_ref[...] = acc + c


_tc_fuse = pl.pallas_call(
    _tc_fuse_body,
    grid=(B // _BLK,),
    in_specs=[
        pl.BlockSpec((_BLK, EMB_DIM), lambda i: (i, 0)),
        pl.BlockSpec((1, 1, _BLK), lambda i: (i, 0, 0)),
        pl.BlockSpec((EMB_DIM, PHYLO_DIM), lambda i: (0, 0)),
        pl.BlockSpec((10, 32), lambda i: (0, 0)),
        pl.BlockSpec((20, 32), lambda i: (0, 0)),
        pl.BlockSpec((30, 32), lambda i: (0, 0)),
        pl.BlockSpec((50, 32), lambda i: (0, 0)),
        pl.BlockSpec((EMB_DIM, FUSED_IN), lambda i: (0, 0)),
        pl.BlockSpec((1, EMB_DIM), lambda i: (0, 0)),
    ],
    out_specs=pl.BlockSpec((_BLK, EMB_DIM), lambda i: (i, 0)),
    out_shape=jax.ShapeDtypeStruct((B, EMB_DIM), jnp.float32),
)


def kernel(species_ids, divergence_times, species_table, phylo_table,
           kingdom_table, phylum_table, class_table, order_table, W, b):
    s_rows = _get_sc_gather()(species_table, species_ids.astype(jnp.int32))
    phylo_pad = jnp.pad(phylo_table, ((0, EMB_DIM - phylo_table.shape[0]),
                                      (0, 0)))
    times = divergence_times.astype(jnp.int32).reshape(B // _BLK, 1, _BLK)
    return _tc_fuse(s_rows, times, phylo_pad, kingdom_table, phylum_table,
                    class_table, order_table, W, b[None, :])
